# Initial kernel scaffold; baseline (speedup 1.0000x reference)
import numpy as np
import jax
import jax.numpy as jnp
from jax.experimental import pallas as pl

_BS = 16
_T = 30
_G = 19
_NB = 5
_NC = 20
_P = _G * _G
_NQ = 26

_ANCW = np.array([1.3221, 3.19275, 5.05587, 9.47112, 11.2364], dtype=np.float32)
_ANCH = np.array([1.73145, 4.00944, 8.09892, 4.84053, 10.0071], dtype=np.float32)

_OBJECT_SCALE = 5.0
_NOOBJECT_SCALE = 1.0
_CLASS_SCALE = 1.0
_COORD_SCALE = 1.0
_PRIOR_SCALE = 0.01
_IOU_THRESH = 0.6

_XF = (np.arange(_P, dtype=np.float32) % _G).reshape(1, 1, _P)
_YF = (np.arange(_P, dtype=np.float32) // _G).reshape(1, 1, _P)
_TRIU = (np.arange(_T)[:, None] <= np.arange(_T)[None, :]).astype(np.float32)
_LT = (np.arange(_T)[None, :] < np.arange(_T)[:, None]).astype(np.float32)


def _yolo_kernel(p_ref, t_ref, o_ref):
    g = jnp.float32(_G)

    p4 = p_ref[...].reshape(_BS, _NB, _NC + 5, _P)
    tx = jax.nn.sigmoid(p4[:, :, 0, :])
    ty = jax.nn.sigmoid(p4[:, :, 1, :])
    tw = p4[:, :, 2, :]
    th = p4[:, :, 3, :]
    conf = jax.nn.sigmoid(p4[:, :, 4, :])
    logits = p4[:, :, 5:, :]

    xf = jnp.asarray(_XF)
    yf = jnp.asarray(_YF)
    awc = jnp.asarray(_ANCW.reshape(1, _NB, 1))
    ahc = jnp.asarray(_ANCH.reshape(1, _NB, 1))

    cx = (tx + xf) / g
    cy = (ty + yf) / g
    cw = jnp.exp(tw) * awc / g
    ch = jnp.exp(th) * ahc / g

    cx1 = cx - cw * 0.5
    cx2 = cx + cw * 0.5
    cy1 = cy - ch * 0.5
    cy2 = cy + ch * 0.5
    a1s = (cw * ch) * _IOU_THRESH

    tg = t_ref[...]
    kls = tg[0]
    bx = tg[1]
    by = tg[2]
    bw = tg[3]
    bh = tg[4]

    zf = (bx == 0.0).astype(jnp.float32)
    zcount = jax.lax.dot_general(
        zf, jnp.asarray(_TRIU), (((1,), (0,)), ((), ())),
        preferred_element_type=jnp.float32)
    valid = zcount == 0.0
    validf = valid.astype(jnp.float32)

    bx1 = bx - bw * 0.5
    bx2 = bx + bw * 0.5
    by1 = by - bh * 0.5
    by2 = by + bh * 0.5
    a2s = jnp.where(valid, bw * bh * _IOU_THRESH, jnp.float32(np.inf))

    any_over = jnp.zeros((_BS, _NB, _P), dtype=jnp.bool_)
    for t in range(_T):
        tbx1 = bx1[:, t][:, None, None]
        tbx2 = bx2[:, t][:, None, None]
        tby1 = by1[:, t][:, None, None]
        tby2 = by2[:, t][:, None, None]
        iw = jnp.maximum(jnp.minimum(cx2, tbx2) - jnp.maximum(cx1, tbx1), 0.0)
        ih = jnp.maximum(jnp.minimum(cy2, tby2) - jnp.maximum(cy1, tby1), 0.0)
        inter = iw * ih
        over = inter * (1.0 + _IOU_THRESH) > a1s + a2s[:, t][:, None, None]
        any_over = jnp.logical_or(any_over, over)
    nof = 1.0 - any_over.astype(jnp.float32)

    s_noobj = jnp.sum(conf * conf * nof)
    s_prior = jnp.sum((tx - 0.5) ** 2 + (ty - 0.5) ** 2 + tw * tw + th * th)

    i_f = jnp.floor(bx * g)
    j_f = jnp.floor(by * g)

    bw3 = bw[:, None, :]
    bh3 = bh[:, None, :]
    awg = awc / g
    ahg = ahc / g
    inter_a = jnp.minimum(awg, bw3) * jnp.minimum(ahg, bh3)
    union_a = awg * ahg + bw3 * bh3 - inter_a
    r_a = inter_a / union_a
    r_max = jnp.max(r_a, axis=1, keepdims=True)
    aiota = jax.lax.broadcasted_iota(jnp.float32, (_BS, _NB, _T), 1)
    a_sel = jnp.min(jnp.where(r_a == r_max, aiota, jnp.float32(_NB)), axis=1)
    onehot_a = (aiota == a_sel[:, None, :]).astype(jnp.float32)
    acw = jnp.sum(onehot_a * awc, axis=1)
    ach = jnp.sum(onehot_a * ahc, axis=1)

    c_cell = a_sel * jnp.float32(_P) + j_f * g + i_f
    same = c_cell[:, :, None] == c_cell[:, None, :]
    prev = jnp.asarray(_LT)[None, :, :] * validf[:, None, :]
    blocked = jnp.max(same.astype(jnp.float32) * prev, axis=2) > 0.0
    applied = jnp.logical_and(valid, jnp.logical_not(blocked))
    appliedf = applied.astype(jnp.float32)

    pieces = []
    for a in range(_NB):
        pieces.extend([
            conf[:, a:a + 1, :], tx[:, a:a + 1, :], ty[:, a:a + 1, :],
            tw[:, a:a + 1, :], th[:, a:a + 1, :], nof[:, a:a + 1, :],
            logits[:, a, :, :],
        ])
    vals = jnp.concatenate(pieces, axis=1)

    p_cell = j_f * g + i_f
    piota = jax.lax.broadcasted_iota(jnp.float32, (_BS, _T, _P), 2)
    oh_p = (piota == p_cell[:, :, None]).astype(jnp.float32)
    r_all = jax.lax.dot_general(
        vals, oh_p, (((2,), (2,)), ((0,), (0,))),
        preferred_element_type=jnp.float32)
    r5 = r_all.reshape(_BS, _NB, _NQ, _T)
    rsel = jnp.sum(r5 * onehot_a[:, :, None, :], axis=1)

    conf_c = rsel[:, 0, :]
    tx_c = rsel[:, 1, :]
    ty_c = rsel[:, 2, :]
    tw_c = rsel[:, 3, :]
    th_c = rsel[:, 4, :]
    nof_c = rsel[:, 5, :]
    logits_c = rsel[:, 6:, :]

    pcx = (tx_c + i_f) / g
    pcy = (ty_c + j_f) / g
    pcw = jnp.exp(tw_c) * acw / g
    pch = jnp.exp(th_c) * ach / g
    p_x1 = pcx - pcw * 0.5
    p_x2 = pcx + pcw * 0.5
    p_y1 = pcy - pch * 0.5
    p_y2 = pcy + pch * 0.5
    iw = jnp.maximum(jnp.minimum(p_x2, bx2) - jnp.maximum(p_x1, bx1), 0.0)
    ih = jnp.maximum(jnp.minimum(p_y2, by2) - jnp.maximum(p_y1, by1), 0.0)
    inter = iw * ih
    union = (p_x2 - p_x1) * (p_y2 - p_y1) + (bx2 - bx1) * (by2 - by1) - inter
    iou_real = inter / union

    ex = bx * g - i_f
    ey = by * g - j_f
    ew = jnp.log(bw * g / acw)
    eh = jnp.log(bh * g / ach)
    sc = 2.0 - bw * bh
    coord_sum = ((tx_c - ex) ** 2 + (ty_c - ey) ** 2
                 + (tw_c - ew) ** 2 + (th_c - eh) ** 2) * (sc * sc)

    m = jnp.max(logits_c, axis=1, keepdims=True)
    e = jnp.exp(logits_c - m)
    cls_prob = e / jnp.sum(e, axis=1, keepdims=True)
    kiota = jax.lax.broadcasted_iota(jnp.float32, (_BS, _NC, _T), 1)
    kls_t = jnp.floor(kls)[:, None, :]
    oh_k = (kiota == kls_t).astype(jnp.float32)
    cls_sum = jnp.sum((cls_prob - oh_k) ** 2, axis=1)

    prior_sum = (tx_c - 0.5) ** 2 + (ty_c - 0.5) ** 2 + tw_c * tw_c + th_c * th_c

    delta = (_OBJECT_SCALE * (conf_c - iou_real) ** 2
             + _CLASS_SCALE * cls_sum
             + _COORD_SCALE * coord_sum
             - _NOOBJECT_SCALE * conf_c * conf_c * nof_c
             - _PRIOR_SCALE * prior_sum)

    total = (_NOOBJECT_SCALE * s_noobj + _PRIOR_SCALE * s_prior
             + jnp.sum(appliedf * delta))
    o_ref[0, 0] = total / jnp.float32(_BS)


def kernel(preds, targets):
    p3 = preds.reshape(_BS, _NB * (_NC + 5), _P)
    tg = jnp.transpose(targets, (2, 0, 1))
    out = pl.pallas_call(
        _yolo_kernel,
        out_shape=jax.ShapeDtypeStruct((1, 1), jnp.float32),
    )(p3, tg)
    return out[0, 0]


# speedup vs baseline: 70.4918x; 70.4918x over previous
import numpy as np
import jax
import jax.numpy as jnp
from jax.experimental import pallas as pl

_BS = 16
_T = 30
_G = 19
_NB = 5
_NC = 20
_P = _G * _G
_NQ = 26

_ANCW = np.array([1.3221, 3.19275, 5.05587, 9.47112, 11.2364], dtype=np.float32)
_ANCH = np.array([1.73145, 4.00944, 8.09892, 4.84053, 10.0071], dtype=np.float32)

_OBJECT_SCALE = 5.0
_NOOBJECT_SCALE = 1.0
_CLASS_SCALE = 1.0
_COORD_SCALE = 1.0
_PRIOR_SCALE = 0.01
_IOU_THRESH = 0.6

_XF = (np.arange(_P, dtype=np.float32) % _G).reshape(1, 1, _P)
_YF = (np.arange(_P, dtype=np.float32) // _G).reshape(1, 1, _P)
_TRIU = (np.arange(_T)[:, None] <= np.arange(_T)[None, :]).astype(np.float32)
_LT = (np.arange(_T)[None, :] < np.arange(_T)[:, None]).astype(np.float32)


def _yolo_kernel(p_ref, t_ref, xy_ref, aw_ref, mk_ref, o_ref):
    g = jnp.float32(_G)

    p4 = p_ref[...].reshape(_BS, _NB, _NC + 5, _P)
    tx = jax.nn.sigmoid(p4[:, :, 0, :])
    ty = jax.nn.sigmoid(p4[:, :, 1, :])
    tw = p4[:, :, 2, :]
    th = p4[:, :, 3, :]
    conf = jax.nn.sigmoid(p4[:, :, 4, :])
    logits = p4[:, :, 5:, :]

    xy = xy_ref[...]
    xf = xy[0:1]
    yf = xy[1:2]
    awh = aw_ref[...]
    awc = awh[0:1].reshape(1, _NB, 1)
    ahc = awh[1:2].reshape(1, _NB, 1)
    mks = mk_ref[...]
    triu = mks[0]
    ltm = mks[1]

    cx = (tx + xf) / g
    cy = (ty + yf) / g
    cw = jnp.exp(tw) * awc / g
    ch = jnp.exp(th) * ahc / g

    cx1 = cx - cw * 0.5
    cx2 = cx + cw * 0.5
    cy1 = cy - ch * 0.5
    cy2 = cy + ch * 0.5
    a1s = (cw * ch) * _IOU_THRESH

    tg = t_ref[...]
    kls = tg[0]
    bx = tg[1]
    by = tg[2]
    bw = tg[3]
    bh = tg[4]

    zf = (bx == 0.0).astype(jnp.float32)
    zcount = jax.lax.dot_general(
        zf, triu, (((1,), (0,)), ((), ())),
        preferred_element_type=jnp.float32)
    valid = zcount == 0.0
    validf = valid.astype(jnp.float32)

    bx1 = bx - bw * 0.5
    bx2 = bx + bw * 0.5
    by1 = by - bh * 0.5
    by2 = by + bh * 0.5
    a2s = jnp.where(valid, bw * bh * _IOU_THRESH, jnp.float32(np.inf))

    any_over = jnp.zeros((_BS, _NB, _P), dtype=jnp.bool_)
    for t in range(_T):
        tbx1 = bx1[:, t][:, None, None]
        tbx2 = bx2[:, t][:, None, None]
        tby1 = by1[:, t][:, None, None]
        tby2 = by2[:, t][:, None, None]
        iw = jnp.maximum(jnp.minimum(cx2, tbx2) - jnp.maximum(cx1, tbx1), 0.0)
        ih = jnp.maximum(jnp.minimum(cy2, tby2) - jnp.maximum(cy1, tby1), 0.0)
        inter = iw * ih
        over = inter * (1.0 + _IOU_THRESH) > a1s + a2s[:, t][:, None, None]
        any_over = jnp.logical_or(any_over, over)
    nof = 1.0 - any_over.astype(jnp.float32)

    s_noobj = jnp.sum(conf * conf * nof)
    s_prior = jnp.sum((tx - 0.5) ** 2 + (ty - 0.5) ** 2 + tw * tw + th * th)

    i_f = jnp.floor(bx * g)
    j_f = jnp.floor(by * g)

    bw3 = bw[:, None, :]
    bh3 = bh[:, None, :]
    awg = awc / g
    ahg = ahc / g
    inter_a = jnp.minimum(awg, bw3) * jnp.minimum(ahg, bh3)
    union_a = awg * ahg + bw3 * bh3 - inter_a
    r_a = inter_a / union_a
    r_max = jnp.max(r_a, axis=1, keepdims=True)
    aiota = jax.lax.broadcasted_iota(jnp.int32, (_BS, _NB, _T), 1).astype(jnp.float32)
    a_sel = jnp.min(jnp.where(r_a == r_max, aiota, jnp.float32(_NB)), axis=1)
    onehot_a = (aiota == a_sel[:, None, :]).astype(jnp.float32)
    acw = jnp.sum(onehot_a * awc, axis=1)
    ach = jnp.sum(onehot_a * ahc, axis=1)

    c_cell = a_sel * jnp.float32(_P) + j_f * g + i_f
    same = c_cell[:, :, None] == c_cell[:, None, :]
    prev = ltm[None, :, :] * validf[:, None, :]
    blocked = jnp.max(same.astype(jnp.float32) * prev, axis=2) > 0.0
    applied = jnp.logical_and(valid, jnp.logical_not(blocked))
    appliedf = applied.astype(jnp.float32)

    pieces = []
    for a in range(_NB):
        pieces.extend([
            conf[:, a:a + 1, :], tx[:, a:a + 1, :], ty[:, a:a + 1, :],
            tw[:, a:a + 1, :], th[:, a:a + 1, :], nof[:, a:a + 1, :],
            logits[:, a, :, :],
        ])
    vals = jnp.concatenate(pieces, axis=1)

    p_cell = j_f * g + i_f
    piota = jax.lax.broadcasted_iota(jnp.int32, (_BS, _T, _P), 2).astype(jnp.float32)
    oh_p = (piota == p_cell[:, :, None]).astype(jnp.float32)
    r_all = jax.lax.dot_general(
        vals, oh_p, (((2,), (2,)), ((0,), (0,))),
        preferred_element_type=jnp.float32)
    r5 = r_all.reshape(_BS, _NB, _NQ, _T)
    rsel = jnp.sum(r5 * onehot_a[:, :, None, :], axis=1)

    conf_c = rsel[:, 0, :]
    tx_c = rsel[:, 1, :]
    ty_c = rsel[:, 2, :]
    tw_c = rsel[:, 3, :]
    th_c = rsel[:, 4, :]
    nof_c = rsel[:, 5, :]
    logits_c = rsel[:, 6:, :]

    pcx = (tx_c + i_f) / g
    pcy = (ty_c + j_f) / g
    pcw = jnp.exp(tw_c) * acw / g
    pch = jnp.exp(th_c) * ach / g
    p_x1 = pcx - pcw * 0.5
    p_x2 = pcx + pcw * 0.5
    p_y1 = pcy - pch * 0.5
    p_y2 = pcy + pch * 0.5
    iw = jnp.maximum(jnp.minimum(p_x2, bx2) - jnp.maximum(p_x1, bx1), 0.0)
    ih = jnp.maximum(jnp.minimum(p_y2, by2) - jnp.maximum(p_y1, by1), 0.0)
    inter = iw * ih
    union = (p_x2 - p_x1) * (p_y2 - p_y1) + (bx2 - bx1) * (by2 - by1) - inter
    iou_real = inter / union

    ex = bx * g - i_f
    ey = by * g - j_f
    ew = jnp.log(bw * g / acw)
    eh = jnp.log(bh * g / ach)
    sc = 2.0 - bw * bh
    coord_sum = ((tx_c - ex) ** 2 + (ty_c - ey) ** 2
                 + (tw_c - ew) ** 2 + (th_c - eh) ** 2) * (sc * sc)

    m = jnp.max(logits_c, axis=1, keepdims=True)
    e = jnp.exp(logits_c - m)
    cls_prob = e / jnp.sum(e, axis=1, keepdims=True)
    kiota = jax.lax.broadcasted_iota(jnp.int32, (_BS, _NC, _T), 1).astype(jnp.float32)
    kls_t = jnp.floor(kls)[:, None, :]
    oh_k = (kiota == kls_t).astype(jnp.float32)
    cls_sum = jnp.sum((cls_prob - oh_k) ** 2, axis=1)

    prior_sum = (tx_c - 0.5) ** 2 + (ty_c - 0.5) ** 2 + tw_c * tw_c + th_c * th_c

    delta = (_OBJECT_SCALE * (conf_c - iou_real) ** 2
             + _CLASS_SCALE * cls_sum
             + _COORD_SCALE * coord_sum
             - _NOOBJECT_SCALE * conf_c * conf_c * nof_c
             - _PRIOR_SCALE * prior_sum)

    total = (_NOOBJECT_SCALE * s_noobj + _PRIOR_SCALE * s_prior
             + jnp.sum(appliedf * delta))
    o_ref[...] = jnp.reshape(total / jnp.float32(_BS), (1, 1))


_XYF = np.stack([_XF.reshape(_P), _YF.reshape(_P)]).reshape(2, 1, _P)
_AWH = np.stack([_ANCW, _ANCH]).reshape(2, _NB, 1)
_MKS = np.stack([_TRIU, _LT])


def kernel(preds, targets):
    p3 = preds.reshape(_BS, _NB * (_NC + 5), _P)
    tg = jnp.transpose(targets, (2, 0, 1))
    out = pl.pallas_call(
        _yolo_kernel,
        out_shape=jax.ShapeDtypeStruct((1, 1), jnp.float32),
    )(p3, tg, jnp.asarray(_XYF), jnp.asarray(_AWH), jnp.asarray(_MKS))
    return out[0, 0]
